# baseline (speedup 1.0000x reference)
import functools

import jax
import jax.numpy as jnp
import numpy as np
from jax import lax
from jax.experimental import pallas as pl
from jax.experimental.pallas import tpu as pltpu
from jax.experimental.pallas import tpu_sc as plsc

_NC = 2
_NS = 16
_NW = _NC * _NS


def _get_emb_np(sin_inp):
    emb = np.stack((np.sin(sin_inp), np.cos(sin_inp)), axis=-1)
    return emb.reshape(sin_inp.shape[0], -1)


@functools.lru_cache(maxsize=None)
def _pos_table_np(org_channels, x, y, z):
    channels = int(np.ceil(org_channels / 6) * 2)
    if channels % 2:
        channels += 1
    inv_freq = (1.0 / (10000.0 ** (np.arange(0, channels, 2, dtype=np.float32)
                                   / np.float32(channels)))).astype(np.float32)
    pos_x = np.arange(x, dtype=np.float32)
    pos_y = np.arange(y, dtype=np.float32)
    pos_z = np.arange(z, dtype=np.float32)
    sin_inp_x = np.einsum('i,j->ij', pos_x, inv_freq)
    sin_inp_y = np.einsum('i,j->ij', pos_y, inv_freq)
    sin_inp_z = np.einsum('i,j->ij', pos_z, inv_freq)
    emb_x = np.broadcast_to(_get_emb_np(sin_inp_x)[:, None, None, :],
                            (x, y, z, channels))
    emb_y = np.broadcast_to(_get_emb_np(sin_inp_y)[None, :, None, :],
                            (x, y, z, channels))
    emb_z = np.broadcast_to(_get_emb_np(sin_inp_z)[None, None, :, :],
                            (x, y, z, channels))
    emb = np.concatenate([emb_x, emb_y, emb_z], axis=-1)
    return np.ascontiguousarray(
        emb[:, :, :, :org_channels].reshape(x * y * z, org_channels)
    ).astype(np.float32)


@functools.lru_cache(maxsize=None)
def _build_sc_gather(N, V, D, L):
    B = N // L
    per_w = N // _NW
    C = 256
    nchunk = per_w // C
    GSUB = C // 128
    CQ = C // 128
    DT = D // 8
    LT = L // 128
    INNER = LT * 8 * 128
    assert per_w % C == 0 and C % 128 == 0 and L % C == 0
    assert nchunk % 2 == 0 and D % 16 == 0

    mesh = plsc.VectorSubcoreMesh(
        core_axis_name="c", subcore_axis_name="s",
        num_cores=_NC, num_subcores=_NS)

    @functools.partial(
        pl.kernel,
        out_type=jax.ShapeDtypeStruct((N * D,), jnp.float32),
        mesh=mesh,
        compiler_params=pltpu.CompilerParams(use_tc_tiling_on_sc=False,
                                             needs_layout_passes=False),
        scratch_types=[
            pltpu.VMEM((2, C), jnp.int32),
            pltpu.VMEM((C, D), jnp.float32),
            pltpu.VMEM((C, D), jnp.float32),
            pltpu.VMEM((DT, CQ * 1024), jnp.float32),
            pltpu.VMEM((DT, CQ * 1024), jnp.float32),
            pltpu.SemaphoreType.DMA,
            pltpu.SemaphoreType.DMA,
            pltpu.SemaphoreType.DMA,
            pltpu.SemaphoreType.DMA,
            pltpu.SemaphoreType.DMA,
            pltpu.SemaphoreType.DMA,
        ],
    )
    def body(idx_hbm, pos_hbm, table_hbm, out_hbm,
             idx_v, rows0, rows1, tb0, tb1, g0, g1, w0, w1, p0, p1):
        cid = lax.axis_index("c")
        sid = lax.axis_index("s")
        wid = sid * _NC + cid
        base = wid * per_w
        rows = (rows0, rows1)
        tbs = (tb0, tb1)
        gsem = (g0, g1)
        wsem = (w0, w1)
        psem = (p0, p1)

        ii = jnp.arange(16, dtype=jnp.int32)
        dtv = []
        dsv = []
        for j in range(D // 16):
            d = 16 * j + ii
            dtv.append(d // 8)
            dsv.append((d % 8) * 128)

        def out_base(k):
            g = base + k * C
            b = g // L
            lt0 = lax.rem(g, L) // 128
            return b * (DT * INNER) + lt0 * 1024

        def fetch_chunk(k, rb):
            g = base + k * C
            pltpu.sync_copy(idx_hbm.at[pl.ds(g, C)], idx_v.at[rb])
            for j in range(GSUB):
                pltpu.async_copy(
                    table_hbm.at[idx_v.at[rb].at[pl.ds(j * 128, 128)]],
                    rows[rb].at[pl.ds(j * 128, 128)], gsem[rb])

        def wait_gather(rb):
            for j in range(GSUB):
                pltpu.make_async_copy(
                    table_hbm.at[idx_v.at[rb].at[pl.ds(j * 128, 128)]],
                    rows[rb].at[pl.ds(j * 128, 128)], gsem[rb]).wait()

        def fire_prefill(k, tb):
            g = base + k * C
            lt0 = lax.rem(g, L) // 128
            pltpu.async_copy(pos_hbm.at[:, pl.ds(lt0 * 1024, CQ * 1024)],
                             tbs[tb], psem[tb])

        def wait_prefill(k, tb):
            g = base + k * C
            lt0 = lax.rem(g, L) // 128
            pltpu.make_async_copy(
                pos_hbm.at[:, pl.ds(lt0 * 1024, CQ * 1024)],
                tbs[tb], psem[tb]).wait()

        def scatter_chunk(rb, tb):
            def _sbody(l, c2):
                s = (l // 128) * 1024 + lax.rem(l, 128)
                for j in range(D // 16):
                    vals = rows[rb][l, pl.ds(16 * j, 16)]
                    plsc.addupdate_scatter(tbs[tb], [dtv[j], dsv[j] + s],
                                           vals)
                return c2
            lax.fori_loop(0, C, _sbody, 0)

        def fire_writeout(k, tb):
            ob = out_base(k)
            for dt in range(DT):
                pltpu.async_copy(tbs[tb].at[dt],
                                 out_hbm.at[pl.ds(ob + dt * INNER, CQ * 1024)],
                                 wsem[tb])

        def wait_writeout(k, tb):
            ob = out_base(k)
            for dt in range(DT):
                pltpu.make_async_copy(
                    tbs[tb].at[dt],
                    out_hbm.at[pl.ds(ob + dt * INNER, CQ * 1024)],
                    wsem[tb]).wait()

        fetch_chunk(0, 0)
        fire_prefill(0, 0)

        def pair_body(i, carry):
            ka = 2 * i

            fetch_chunk(ka + 1, 1)

            @pl.when(i > 0)
            def _():
                wait_writeout(ka - 1, 1)
            fire_prefill(ka + 1, 1)

            wait_gather(0)
            wait_prefill(ka, 0)
            scatter_chunk(0, 0)
            fire_writeout(ka, 0)

            @pl.when(i < nchunk // 2 - 1)
            def _():
                fetch_chunk(ka + 2, 0)
                wait_writeout(ka, 0)
                fire_prefill(ka + 2, 0)

            wait_gather(1)
            wait_prefill(ka + 1, 1)
            scatter_chunk(1, 1)
            fire_writeout(ka + 1, 1)
            return carry

        lax.fori_loop(0, nchunk // 2, pair_body, 0)
        wait_writeout(nchunk - 2, 0)
        wait_writeout(nchunk - 1, 1)

    return body


def kernel(x, W):
    B, L1, L2, orbit = x.shape
    V, D = W.shape
    L = L1 * L2 * orbit
    N = B * L
    DT, LT = D // 8, L // 128
    flat_idx = x.reshape(N)
    pos = _pos_table_np(D, L1, L2, orbit)
    pos_tiles = np.ascontiguousarray(
        pos.T.reshape(DT, 8, LT, 128).transpose(0, 2, 1, 3)
    ).reshape(DT, LT * 8 * 128)
    out = _build_sc_gather(N, V, D, L)(flat_idx, jnp.asarray(pos_tiles), W)
    out5 = out.reshape(B, DT, LT, 8, 128)
    return out5.transpose(0, 2, 4, 1, 3).reshape(B, L, D)


# speedup vs baseline: 1.1838x; 1.1838x over previous
import functools

import jax
import jax.numpy as jnp
import numpy as np
from jax import lax
from jax.experimental import pallas as pl
from jax.experimental.pallas import tpu as pltpu
from jax.experimental.pallas import tpu_sc as plsc

_NC = 2
_NS = 16
_NW = _NC * _NS


def _get_emb_np(sin_inp):
    emb = np.stack((np.sin(sin_inp), np.cos(sin_inp)), axis=-1)
    return emb.reshape(sin_inp.shape[0], -1)


@functools.lru_cache(maxsize=None)
def _pos_table_np(org_channels, x, y, z):
    channels = int(np.ceil(org_channels / 6) * 2)
    if channels % 2:
        channels += 1
    inv_freq = (1.0 / (10000.0 ** (np.arange(0, channels, 2, dtype=np.float32)
                                   / np.float32(channels)))).astype(np.float32)
    pos_x = np.arange(x, dtype=np.float32)
    pos_y = np.arange(y, dtype=np.float32)
    pos_z = np.arange(z, dtype=np.float32)
    sin_inp_x = np.einsum('i,j->ij', pos_x, inv_freq)
    sin_inp_y = np.einsum('i,j->ij', pos_y, inv_freq)
    sin_inp_z = np.einsum('i,j->ij', pos_z, inv_freq)
    emb_x = np.broadcast_to(_get_emb_np(sin_inp_x)[:, None, None, :],
                            (x, y, z, channels))
    emb_y = np.broadcast_to(_get_emb_np(sin_inp_y)[None, :, None, :],
                            (x, y, z, channels))
    emb_z = np.broadcast_to(_get_emb_np(sin_inp_z)[None, None, :, :],
                            (x, y, z, channels))
    emb = np.concatenate([emb_x, emb_y, emb_z], axis=-1)
    return np.ascontiguousarray(
        emb[:, :, :, :org_channels].reshape(x * y * z, org_channels)
    ).astype(np.float32)


@functools.lru_cache(maxsize=None)
def _build_sc_gather(N, V, D, L):
    B = N // L
    per_w = N // _NW
    C = 256
    nchunk = per_w // C
    GSUB = C // 128
    CQ = C // 128
    DT = D // 8
    LT = L // 128
    INNER = LT * 8 * 128
    assert per_w % C == 0 and C % 128 == 0 and L % C == 0
    assert nchunk % 2 == 0 and D % 16 == 0

    mesh = plsc.VectorSubcoreMesh(
        core_axis_name="c", subcore_axis_name="s",
        num_cores=_NC, num_subcores=_NS)

    @functools.partial(
        pl.kernel,
        out_type=jax.ShapeDtypeStruct((N * D,), jnp.float32),
        mesh=mesh,
        compiler_params=pltpu.CompilerParams(use_tc_tiling_on_sc=False,
                                             needs_layout_passes=False),
        scratch_types=[
            pltpu.VMEM((2, C), jnp.int32),
            pltpu.VMEM((C, D), jnp.float32),
            pltpu.VMEM((C, D), jnp.float32),
            pltpu.VMEM((DT * CQ * 1024,), jnp.float32),
            pltpu.VMEM((DT * CQ * 1024,), jnp.float32),
            pltpu.SemaphoreType.DMA,
            pltpu.SemaphoreType.DMA,
            pltpu.SemaphoreType.DMA,
            pltpu.SemaphoreType.DMA,
            pltpu.SemaphoreType.DMA,
            pltpu.SemaphoreType.DMA,
        ],
    )
    def body(idx_hbm, pos_hbm, table_hbm, out_hbm,
             idx_v, rows0, rows1, tb0, tb1, g0, g1, w0, w1, p0, p1):
        cid = lax.axis_index("c")
        sid = lax.axis_index("s")
        wid = sid * _NC + cid
        base = wid * per_w
        rows = (rows0, rows1)
        tbs = (tb0, tb1)
        gsem = (g0, g1)
        wsem = (w0, w1)
        psem = (p0, p1)

        ii = jnp.arange(16, dtype=jnp.int32)
        cvec = []
        for j in range(D // 16):
            d = 16 * j + ii
            cvec.append((d // 8) * (CQ * 1024) + (d % 8) * 128)

        def out_base(k):
            g = base + k * C
            b = g // L
            lt0 = lax.rem(g, L) // 128
            return b * (DT * INNER) + lt0 * 1024

        def fetch_chunk(k, rb):
            g = base + k * C
            pltpu.sync_copy(idx_hbm.at[pl.ds(g, C)], idx_v.at[rb])
            for j in range(GSUB):
                pltpu.async_copy(
                    table_hbm.at[idx_v.at[rb].at[pl.ds(j * 128, 128)]],
                    rows[rb].at[pl.ds(j * 128, 128)], gsem[rb])

        def wait_gather(rb):
            for j in range(GSUB):
                pltpu.make_async_copy(
                    table_hbm.at[idx_v.at[rb].at[pl.ds(j * 128, 128)]],
                    rows[rb].at[pl.ds(j * 128, 128)], gsem[rb]).wait()

        def fire_prefill(k, tb):
            lb = lax.rem(k, L // C)
            pltpu.async_copy(pos_hbm.at[lb], tbs[tb], psem[tb])

        def wait_prefill(k, tb):
            lb = lax.rem(k, L // C)
            pltpu.make_async_copy(pos_hbm.at[lb], tbs[tb], psem[tb]).wait()

        def scatter_chunk(rb, tb):
            @plsc.parallel_loop(0, C, unroll=4)
            def _(l):
                s = (l // 128) * 1024 + lax.rem(l, 128)
                bs = jnp.broadcast_to(s, (16,))
                for j in range(D // 16):
                    vals = rows[rb][l, pl.ds(16 * j, 16)]
                    plsc.addupdate_scatter(tbs[tb], [cvec[j] + bs], vals)

        def fire_writeout(k, tb):
            ob = out_base(k)
            for dt in range(DT):
                pltpu.async_copy(tbs[tb].at[pl.ds(dt * CQ * 1024, CQ * 1024)],
                                 out_hbm.at[pl.ds(ob + dt * INNER, CQ * 1024)],
                                 wsem[tb])

        def wait_writeout(k, tb):
            ob = out_base(k)
            for dt in range(DT):
                pltpu.make_async_copy(
                    tbs[tb].at[pl.ds(dt * CQ * 1024, CQ * 1024)],
                    out_hbm.at[pl.ds(ob + dt * INNER, CQ * 1024)],
                    wsem[tb]).wait()

        fetch_chunk(0, 0)
        fire_prefill(0, 0)

        def pair_body(i, carry):
            ka = 2 * i

            fetch_chunk(ka + 1, 1)

            @pl.when(i > 0)
            def _():
                wait_writeout(ka - 1, 1)
            fire_prefill(ka + 1, 1)

            wait_gather(0)
            wait_prefill(ka, 0)
            scatter_chunk(0, 0)
            fire_writeout(ka, 0)

            @pl.when(i < nchunk // 2 - 1)
            def _():
                fetch_chunk(ka + 2, 0)
                wait_writeout(ka, 0)
                fire_prefill(ka + 2, 0)

            wait_gather(1)
            wait_prefill(ka + 1, 1)
            scatter_chunk(1, 1)
            fire_writeout(ka + 1, 1)
            return carry

        lax.fori_loop(0, nchunk // 2, pair_body, 0)
        wait_writeout(nchunk - 2, 0)
        wait_writeout(nchunk - 1, 1)

    return body


def kernel(x, W):
    B, L1, L2, orbit = x.shape
    V, D = W.shape
    L = L1 * L2 * orbit
    N = B * L
    DT, LT = D // 8, L // 128
    flat_idx = x.reshape(N)
    CQ = 256 // 128
    pos = _pos_table_np(D, L1, L2, orbit)
    pos_tiles = np.ascontiguousarray(
        pos.T.reshape(DT, 8, LT // CQ, CQ, 128).transpose(2, 0, 3, 1, 4)
    ).reshape(LT // CQ, DT * CQ * 1024)
    out = _build_sc_gather(N, V, D, L)(flat_idx, jnp.asarray(pos_tiles), W)
    out5 = out.reshape(B, DT, LT, 8, 128)
    return out5.transpose(0, 2, 4, 1, 3).reshape(B, L, D)


# speedup vs baseline: 2.3820x; 2.0121x over previous
import functools

import jax
import jax.numpy as jnp
import numpy as np
from jax import lax
from jax.experimental import pallas as pl
from jax.experimental.pallas import tpu as pltpu
from jax.experimental.pallas import tpu_sc as plsc

_NC = 2
_NS = 16
_NW = _NC * _NS


def _get_emb_np(sin_inp):
    emb = np.stack((np.sin(sin_inp), np.cos(sin_inp)), axis=-1)
    return emb.reshape(sin_inp.shape[0], -1)


@functools.lru_cache(maxsize=None)
def _pos_table_np(org_channels, x, y, z):
    channels = int(np.ceil(org_channels / 6) * 2)
    if channels % 2:
        channels += 1
    inv_freq = (1.0 / (10000.0 ** (np.arange(0, channels, 2, dtype=np.float32)
                                   / np.float32(channels)))).astype(np.float32)
    pos_x = np.arange(x, dtype=np.float32)
    pos_y = np.arange(y, dtype=np.float32)
    pos_z = np.arange(z, dtype=np.float32)
    sin_inp_x = np.einsum('i,j->ij', pos_x, inv_freq)
    sin_inp_y = np.einsum('i,j->ij', pos_y, inv_freq)
    sin_inp_z = np.einsum('i,j->ij', pos_z, inv_freq)
    emb_x = np.broadcast_to(_get_emb_np(sin_inp_x)[:, None, None, :],
                            (x, y, z, channels))
    emb_y = np.broadcast_to(_get_emb_np(sin_inp_y)[None, :, None, :],
                            (x, y, z, channels))
    emb_z = np.broadcast_to(_get_emb_np(sin_inp_z)[None, None, :, :],
                            (x, y, z, channels))
    emb = np.concatenate([emb_x, emb_y, emb_z], axis=-1)
    return np.ascontiguousarray(
        emb[:, :, :, :org_channels].reshape(x * y * z, org_channels)
    ).astype(np.float32)


@functools.lru_cache(maxsize=None)
def _build_sc_gather(N, V, D, L):
    B = N // L
    per_w = N // _NW
    C = 256
    nchunk = per_w // C
    GSUB = C // 128
    CQ = C // 128
    DT = D // 8
    LT = L // 128
    INNER = LT * 8 * 128
    assert per_w % C == 0 and C % 128 == 0 and L % C == 0
    assert nchunk % 2 == 0 and D % 16 == 0

    mesh = plsc.VectorSubcoreMesh(
        core_axis_name="c", subcore_axis_name="s",
        num_cores=_NC, num_subcores=_NS)

    @functools.partial(
        pl.kernel,
        out_type=jax.ShapeDtypeStruct((N * D // 128, 128), jnp.float32),
        mesh=mesh,
        compiler_params=pltpu.CompilerParams(use_tc_tiling_on_sc=False,
                                             needs_layout_passes=False),
        scratch_types=[
            pltpu.VMEM((2, C), jnp.int32),
            pltpu.VMEM((C, D), jnp.float32),
            pltpu.VMEM((C, D), jnp.float32),
            pltpu.VMEM((D, C + 1), jnp.float32),
            pltpu.VMEM((D, C + 1), jnp.float32),
            pltpu.SemaphoreType.DMA,
            pltpu.SemaphoreType.DMA,
            pltpu.SemaphoreType.DMA,
            pltpu.SemaphoreType.DMA,
            pltpu.SemaphoreType.DMA,
            pltpu.SemaphoreType.DMA,
        ],
    )
    def body(idx_hbm, pos_hbm, table_hbm, out_hbm,
             idx_v, rows0, rows1, tb0, tb1, g0, g1, w0, w1, p0, p1):
        cid = lax.axis_index("c")
        sid = lax.axis_index("s")
        wid = sid * _NC + cid
        base = wid * per_w
        rows = (rows0, rows1)
        tbs = (tb0, tb1)
        gsem = (g0, g1)
        wsem = (w0, w1)
        psem = (p0, p1)

        ii = jnp.arange(16, dtype=jnp.int32)
        dvec = [16 * j + ii for j in range(D // 16)]

        def out_base128(k):
            g = base + k * C
            b = g // L
            lt0 = lax.rem(g, L) // 128
            return b * (DT * INNER // 128) + lt0 * 8

        def fetch_chunk(k, rb):
            g = base + k * C
            pltpu.sync_copy(idx_hbm.at[pl.ds(g, C)], idx_v.at[rb])
            for j in range(GSUB):
                pltpu.async_copy(
                    table_hbm.at[idx_v.at[rb].at[pl.ds(j * 128, 128)]],
                    rows[rb].at[pl.ds(j * 128, 128)], gsem[rb])

        def wait_gather(rb):
            for j in range(GSUB):
                pltpu.make_async_copy(
                    table_hbm.at[idx_v.at[rb].at[pl.ds(j * 128, 128)]],
                    rows[rb].at[pl.ds(j * 128, 128)], gsem[rb]).wait()

        def fire_prefill(k, tb):
            lb = lax.rem(k, L // C)
            pltpu.async_copy(pos_hbm.at[lb], tbs[tb], psem[tb])

        def wait_prefill(k, tb):
            lb = lax.rem(k, L // C)
            pltpu.make_async_copy(pos_hbm.at[lb], tbs[tb], psem[tb]).wait()

        def scatter_chunk(rb, tb):
            @plsc.parallel_loop(0, C, unroll=4)
            def _(l):
                bs = jnp.broadcast_to(l, (16,))
                for j in range(D // 16):
                    vals = rows[rb][l, pl.ds(16 * j, 16)]
                    plsc.addupdate_scatter(tbs[tb], [dvec[j], bs], vals)

        def fire_writeout(k, tb):
            ob = out_base128(k)
            for dt in range(DT):
                for lq in range(CQ):
                    pltpu.async_copy(
                        tbs[tb].at[pl.ds(dt * 8, 8), pl.ds(lq * 128, 128)],
                        out_hbm.at[pl.ds(ob + dt * (INNER // 128) + lq * 8, 8)],
                        wsem[tb])

        def wait_writeout(k, tb):
            ob = out_base128(k)
            for dt in range(DT):
                for lq in range(CQ):
                    pltpu.make_async_copy(
                        tbs[tb].at[pl.ds(dt * 8, 8), pl.ds(lq * 128, 128)],
                        out_hbm.at[pl.ds(ob + dt * (INNER // 128) + lq * 8, 8)],
                        wsem[tb]).wait()

        fetch_chunk(0, 0)
        fire_prefill(0, 0)

        def pair_body(i, carry):
            ka = 2 * i

            fetch_chunk(ka + 1, 1)

            @pl.when(i > 0)
            def _():
                wait_writeout(ka - 1, 1)
            fire_prefill(ka + 1, 1)

            wait_gather(0)
            wait_prefill(ka, 0)
            scatter_chunk(0, 0)
            fire_writeout(ka, 0)

            @pl.when(i < nchunk // 2 - 1)
            def _():
                fetch_chunk(ka + 2, 0)
                wait_writeout(ka, 0)
                fire_prefill(ka + 2, 0)

            wait_gather(1)
            wait_prefill(ka + 1, 1)
            scatter_chunk(1, 1)
            fire_writeout(ka + 1, 1)
            return carry

        lax.fori_loop(0, nchunk // 2, pair_body, 0)
        wait_writeout(nchunk - 2, 0)
        wait_writeout(nchunk - 1, 1)

    return body


def kernel(x, W):
    B, L1, L2, orbit = x.shape
    V, D = W.shape
    L = L1 * L2 * orbit
    N = B * L
    DT, LT = D // 8, L // 128
    flat_idx = x.reshape(N)
    C = 256
    pos = _pos_table_np(D, L1, L2, orbit)
    pos_pad = np.zeros((L // C, D, C + 1), np.float32)
    pos_pad[:, :, :C] = pos.reshape(L // C, C, D).transpose(0, 2, 1)
    out = _build_sc_gather(N, V, D, L)(flat_idx, jnp.asarray(pos_pad), W)
    out5 = out.reshape(B, DT, LT, 8, 128)
    return out5.transpose(0, 2, 4, 1, 3).reshape(B, L, D)


# speedup vs baseline: 3.0230x; 1.2691x over previous
import functools

import jax
import jax.numpy as jnp
import numpy as np
from jax import lax
from jax.experimental import pallas as pl
from jax.experimental.pallas import tpu as pltpu
from jax.experimental.pallas import tpu_sc as plsc

_NC = 2
_NS = 16
_NW = _NC * _NS


def _get_emb_np(sin_inp):
    emb = np.stack((np.sin(sin_inp), np.cos(sin_inp)), axis=-1)
    return emb.reshape(sin_inp.shape[0], -1)


@functools.lru_cache(maxsize=None)
def _pos_table_np(org_channels, x, y, z):
    channels = int(np.ceil(org_channels / 6) * 2)
    if channels % 2:
        channels += 1
    inv_freq = (1.0 / (10000.0 ** (np.arange(0, channels, 2, dtype=np.float32)
                                   / np.float32(channels)))).astype(np.float32)
    pos_x = np.arange(x, dtype=np.float32)
    pos_y = np.arange(y, dtype=np.float32)
    pos_z = np.arange(z, dtype=np.float32)
    sin_inp_x = np.einsum('i,j->ij', pos_x, inv_freq)
    sin_inp_y = np.einsum('i,j->ij', pos_y, inv_freq)
    sin_inp_z = np.einsum('i,j->ij', pos_z, inv_freq)
    emb_x = np.broadcast_to(_get_emb_np(sin_inp_x)[:, None, None, :],
                            (x, y, z, channels))
    emb_y = np.broadcast_to(_get_emb_np(sin_inp_y)[None, :, None, :],
                            (x, y, z, channels))
    emb_z = np.broadcast_to(_get_emb_np(sin_inp_z)[None, None, :, :],
                            (x, y, z, channels))
    emb = np.concatenate([emb_x, emb_y, emb_z], axis=-1)
    return np.ascontiguousarray(
        emb[:, :, :, :org_channels].reshape(x * y * z, org_channels)
    ).astype(np.float32)


@functools.lru_cache(maxsize=None)
def _build_sc_gather(N, V, D, L):
    B = N // L
    per_w = N // _NW
    C = 128
    nchunk = per_w // C
    GSUB = C // 128
    CQ = C // 128
    DT = D // 8
    LT = L // 128
    INNER = LT * 8 * 128
    assert per_w % C == 0 and C % 128 == 0 and L % C == 0
    assert nchunk % 2 == 0 and D % 16 == 0

    mesh = plsc.VectorSubcoreMesh(
        core_axis_name="c", subcore_axis_name="s",
        num_cores=_NC, num_subcores=_NS)

    @functools.partial(
        pl.kernel,
        out_type=jax.ShapeDtypeStruct((N * D // 128, 128), jnp.float32),
        mesh=mesh,
        compiler_params=pltpu.CompilerParams(use_tc_tiling_on_sc=False,
                                             needs_layout_passes=False),
        scratch_types=[
            pltpu.VMEM((2, C), jnp.int32),
            pltpu.VMEM((C, D), jnp.float32),
            pltpu.VMEM((C, D), jnp.float32),
            pltpu.VMEM((D, C + 1), jnp.float32),
            pltpu.VMEM((D, C + 1), jnp.float32),
            pltpu.VMEM((L, D), jnp.float32),
            pltpu.SemaphoreType.DMA,
            pltpu.SemaphoreType.DMA,
            pltpu.SemaphoreType.DMA,
            pltpu.SemaphoreType.DMA,
        ],
    )
    def body(idx_hbm, pos_hbm, table_hbm, out_hbm,
             idx_v, rows0, rows1, tb0, tb1, pos_v, g0, g1, w0, w1):
        cid = lax.axis_index("c")
        sid = lax.axis_index("s")
        wid = sid * _NC + cid
        base = wid * per_w
        rows = (rows0, rows1)
        tbs = (tb0, tb1)
        gsem = (g0, g1)
        wsem = (w0, w1)
        pltpu.sync_copy(pos_hbm, pos_v)

        ii = jnp.arange(16, dtype=jnp.int32)
        dvec = [16 * j + ii for j in range(D // 16)]

        def out_base128(k):
            g = base + k * C
            b = g // L
            lt0 = lax.rem(g, L) // 128
            return b * (DT * INNER // 128) + lt0 * 8

        def fetch_chunk(k, rb):
            g = base + k * C
            pltpu.sync_copy(idx_hbm.at[pl.ds(g, C)], idx_v.at[rb])
            for j in range(GSUB):
                pltpu.async_copy(
                    table_hbm.at[idx_v.at[rb].at[pl.ds(j * 128, 128)]],
                    rows[rb].at[pl.ds(j * 128, 128)], gsem[rb])

        def wait_gather(rb):
            for j in range(GSUB):
                pltpu.make_async_copy(
                    table_hbm.at[idx_v.at[rb].at[pl.ds(j * 128, 128)]],
                    rows[rb].at[pl.ds(j * 128, 128)], gsem[rb]).wait()

        def scatter_chunk(k, rb, tb):
            po = lax.rem(k, L // C) * C

            @plsc.parallel_loop(0, C, unroll=4)
            def _(l):
                bs = jnp.broadcast_to(l, (16,))
                for j in range(D // 16):
                    sl = pl.ds(16 * j, 16)
                    vals = rows[rb][l, sl] + pos_v[po + l, sl]
                    plsc.store_scatter(tbs[tb], [dvec[j], bs], vals)

        def fire_writeout(k, tb):
            ob = out_base128(k)
            for dt in range(DT):
                for lq in range(CQ):
                    pltpu.async_copy(
                        tbs[tb].at[pl.ds(dt * 8, 8), pl.ds(lq * 128, 128)],
                        out_hbm.at[pl.ds(ob + dt * (INNER // 128) + lq * 8, 8)],
                        wsem[tb])

        def wait_writeout(k, tb):
            ob = out_base128(k)
            for dt in range(DT):
                for lq in range(CQ):
                    pltpu.make_async_copy(
                        tbs[tb].at[pl.ds(dt * 8, 8), pl.ds(lq * 128, 128)],
                        out_hbm.at[pl.ds(ob + dt * (INNER // 128) + lq * 8, 8)],
                        wsem[tb]).wait()

        fetch_chunk(0, 0)

        def pair_body(i, carry):
            ka = 2 * i

            fetch_chunk(ka + 1, 1)

            wait_gather(0)

            @pl.when(i > 0)
            def _():
                wait_writeout(ka - 2, 0)
            scatter_chunk(ka, 0, 0)
            fire_writeout(ka, 0)

            @pl.when(i < nchunk // 2 - 1)
            def _():
                fetch_chunk(ka + 2, 0)

            wait_gather(1)

            @pl.when(i > 0)
            def _():
                wait_writeout(ka - 1, 1)
            scatter_chunk(ka + 1, 1, 1)
            fire_writeout(ka + 1, 1)
            return carry

        lax.fori_loop(0, nchunk // 2, pair_body, 0)
        wait_writeout(nchunk - 2, 0)
        wait_writeout(nchunk - 1, 1)

    return body


def kernel(x, W):
    B, L1, L2, orbit = x.shape
    V, D = W.shape
    L = L1 * L2 * orbit
    N = B * L
    DT, LT = D // 8, L // 128
    flat_idx = x.reshape(N)
    pos = jnp.asarray(_pos_table_np(D, L1, L2, orbit))
    out = _build_sc_gather(N, V, D, L)(flat_idx, pos, W)
    out5 = out.reshape(B, DT, LT, 8, 128)
    return out5.transpose(0, 2, 4, 1, 3).reshape(B, L, D)
